# baseline (speedup 1.0000x reference)
import functools

import jax
import jax.numpy as jnp
from jax.experimental import pallas as pl
from jax.experimental.pallas import tpu as pltpu
from jax.experimental.pallas import tpu_sc as plsc

BB, DD, HH, KK = 8192, 1024, 256, 8
TILE = 256
NT = BB // TILE
SPLIT = 4


def _classify_body(x_ref, wc_ref, bc_ref, a_ref, rank_ref, cnt_ref):
    x_t = x_ref[...]
    logits = jnp.dot(x_t, wc_ref[...], preferred_element_type=jnp.float32)
    logits = logits + bc_ref[...]
    m = jnp.max(logits, axis=1, keepdims=True)
    lane = jax.lax.broadcasted_iota(jnp.int32, (TILE, KK), 1)
    amax = jnp.min(jnp.where(logits == m, lane, KK), axis=1, keepdims=True)
    onehot = (lane == amax).astype(jnp.float32)
    ri = jax.lax.broadcasted_iota(jnp.int32, (TILE, TILE), 0)
    ci = jax.lax.broadcasted_iota(jnp.int32, (TILE, TILE), 1)
    ltri = (ci < ri).astype(jnp.float32)
    ranks = jax.lax.dot(ltri, onehot,
                        precision=jax.lax.Precision.HIGHEST)
    rank_tok = jnp.sum(ranks * onehot, axis=1, keepdims=True)
    a_ref[...] = amax
    rank_ref[...] = rank_tok
    cnt_ref[...] = jnp.sum(onehot, axis=0, keepdims=True)[None]


def _classify(x, wc, bc):
    return pl.pallas_call(
        _classify_body,
        grid=(NT,),
        in_specs=[
            pl.BlockSpec((TILE, DD), lambda t: (t, 0)),
            pl.BlockSpec((DD, KK), lambda t: (0, 0)),
            pl.BlockSpec((1, KK), lambda t: (0, 0)),
        ],
        out_specs=[
            pl.BlockSpec((TILE, 1), lambda t: (t, 0)),
            pl.BlockSpec((TILE, 1), lambda t: (t, 0)),
            pl.BlockSpec((1, 1, KK), lambda t: (t, 0, 0)),
        ],
        out_shape=[
            jax.ShapeDtypeStruct((BB, 1), jnp.int32),
            jax.ShapeDtypeStruct((BB, 1), jnp.float32),
            jax.ShapeDtypeStruct((NT, 1, KK), jnp.float32),
        ],
    )(x, wc, bc)


def _destiny_body(a_ref, rank_ref, cnt_ref, dest4_ref, offs_ref):
    t = pl.program_id(0)
    cnts = cnt_ref[:, 0, :]
    tidx = jax.lax.broadcasted_iota(jnp.int32, (NT, KK), 0)
    carry = jnp.sum(jnp.where(tidx < t, cnts, 0.0), axis=0, keepdims=True)
    tot = jnp.sum(cnts, axis=0, keepdims=True)
    er = jax.lax.broadcasted_iota(jnp.int32, (KK, KK), 0)
    ec = jax.lax.broadcasted_iota(jnp.int32, (KK, KK), 1)
    xtri = (er < ec).astype(jnp.float32)
    offs = jax.lax.dot(tot, xtri,
                       precision=jax.lax.Precision.HIGHEST)
    v = offs + carry
    a_t = a_ref[...]
    lane = jax.lax.broadcasted_iota(jnp.int32, (TILE, KK), 1)
    onehot = (lane == a_t).astype(jnp.float32)
    base = jnp.sum(onehot * v, axis=1, keepdims=True)
    dest = (base + rank_ref[...]).astype(jnp.int32)
    sub = jax.lax.broadcasted_iota(jnp.int32, (TILE, SPLIT), 1)
    dest4_ref[...] = dest * SPLIT + sub
    offs_ref[...] = offs.astype(jnp.int32)


def _destiny(a, rank, cnt):
    return pl.pallas_call(
        _destiny_body,
        grid=(NT,),
        in_specs=[
            pl.BlockSpec((TILE, 1), lambda t: (t, 0)),
            pl.BlockSpec((TILE, 1), lambda t: (t, 0)),
            pl.BlockSpec((NT, 1, KK), lambda t: (0, 0, 0)),
        ],
        out_specs=[
            pl.BlockSpec((TILE, SPLIT), lambda t: (t, 0)),
            pl.BlockSpec((1, KK), lambda t: (0, 0)),
        ],
        out_shape=[
            jax.ShapeDtypeStruct((BB, SPLIT), jnp.int32),
            jax.ShapeDtypeStruct((1, KK), jnp.int32),
        ],
    )(a, rank, cnt)


_SC_WIN = 128
NROWS = BB * SPLIT
SUBD = DD // SPLIT


def _sc_scatter(x4, dest_row):
    mesh = plsc.VectorSubcoreMesh(core_axis_name="core",
                                  subcore_axis_name="subcore")

    @functools.partial(
        pl.kernel,
        out_type=jax.ShapeDtypeStruct((NROWS, SUBD), jnp.float32),
        mesh=mesh)
    def run(x_hbm, i_hbm, o_hbm):
        def body(x_vmem, i_vmem):
            pltpu.sync_copy(x_vmem, o_hbm.at[i_vmem.at[0]])

        pltpu.emit_pipeline(
            body,
            grid=(NROWS // _SC_WIN,),
            in_specs=[
                pl.BlockSpec((_SC_WIN, SUBD), lambda i: (i, 0)),
                pl.BlockSpec((1, _SC_WIN), lambda i: (0, i)),
            ],
            out_specs=[],
            core_axis_name=("core", "subcore"),
            dimension_semantics=(pltpu.PARALLEL,),
        )(x_hbm, i_hbm)

    return run(x4, dest_row)


def _sc_gather(recon4, dest_row):
    mesh = plsc.VectorSubcoreMesh(core_axis_name="core",
                                  subcore_axis_name="subcore")

    @functools.partial(
        pl.kernel,
        out_type=jax.ShapeDtypeStruct((NROWS, SUBD), jnp.float32),
        mesh=mesh)
    def run(r_hbm, i_hbm, o_hbm):
        def body(i_vmem, o_vmem):
            pltpu.sync_copy(r_hbm.at[i_vmem.at[0]], o_vmem)

        pltpu.emit_pipeline(
            body,
            grid=(NROWS // _SC_WIN,),
            in_specs=[pl.BlockSpec((1, _SC_WIN), lambda i: (0, i))],
            out_specs=[pl.BlockSpec((_SC_WIN, SUBD), lambda i: (i, 0))],
            core_axis_name=("core", "subcore"),
            dimension_semantics=(pltpu.PARALLEL,),
        )(i_hbm, o_hbm)

    return run(recon4, dest_row)


def _ae_body(offs_ref, x_ref, w1_ref, b1_ref, w2_ref, b2_ref, o_ref):
    t = pl.program_id(0)
    row0 = t * TILE
    x_t = x_ref[...]
    rows = jax.lax.broadcasted_iota(jnp.int32, (TILE, 1), 0)
    for e in range(KK):
        s = jnp.clip(offs_ref[e] - row0, 0, TILE)
        en = jnp.clip(offs_ref[e + 1] - row0, 0, TILE)

        @pl.when(en > s)
        def _():
            h = jnp.dot(x_t, w1_ref[e], preferred_element_type=jnp.float32)
            h = jax.nn.relu(h + b1_ref[e][None, :])
            r = jnp.dot(h, w2_ref[e], preferred_element_type=jnp.float32)
            r = r + b2_ref[e][None, :]
            mask = (rows >= s) & (rows < en)
            o_ref[...] = jnp.where(mask, r, o_ref[...])


def _grouped_ae(offs9, xs, w1, b1, w2, b2):
    grid_spec = pltpu.PrefetchScalarGridSpec(
        num_scalar_prefetch=1,
        grid=(NT,),
        in_specs=[
            pl.BlockSpec((TILE, DD), lambda t, offs: (t, 0)),
            pl.BlockSpec((KK, DD, HH), lambda t, offs: (0, 0, 0)),
            pl.BlockSpec((KK, HH), lambda t, offs: (0, 0)),
            pl.BlockSpec((KK, HH, DD), lambda t, offs: (0, 0, 0)),
            pl.BlockSpec((KK, DD), lambda t, offs: (0, 0)),
        ],
        out_specs=pl.BlockSpec((TILE, DD), lambda t, offs: (t, 0)),
    )
    return pl.pallas_call(
        _ae_body,
        grid_spec=grid_spec,
        out_shape=jax.ShapeDtypeStruct((BB, DD), jnp.float32),
    )(offs9, xs, w1, b1, w2, b2)


def kernel(x, W1, b1, W2, b2, Wc, bc):
    a, rank, cnt = _classify(x, Wc, bc.reshape(1, KK))
    dest4, offs = _destiny(a, rank, cnt)
    offs9 = jnp.concatenate(
        [offs.reshape(KK), jnp.array([BB], jnp.int32)])
    dest_row = dest4.reshape(1, NROWS)
    xs4 = _sc_scatter(x.reshape(NROWS, SUBD), dest_row)
    recon = _grouped_ae(offs9, xs4.reshape(BB, DD), W1, b1, W2, b2)
    x_out = _sc_gather(recon.reshape(NROWS, SUBD), dest_row)
    return (x_out.reshape(BB, DD), a.reshape(BB))


# speedup vs baseline: 1.4332x; 1.4332x over previous
import functools

import jax
import jax.numpy as jnp
from jax.experimental import pallas as pl
from jax.experimental.pallas import tpu as pltpu
from jax.experimental.pallas import tpu_sc as plsc

BB, DD, HH, KK = 8192, 1024, 256, 8
TILE = 256
NT = BB // TILE
SPLIT = 4
SUBD = DD // SPLIT
NROWS = BB * SPLIT


def _classify_body(x_ref, wc_ref, bc_ref, a_ref, rank_ref, cnt_ref):
    x_t = x_ref[...]
    logits = jnp.dot(x_t, wc_ref[...], preferred_element_type=jnp.float32)
    logits = logits + bc_ref[...]
    m = jnp.max(logits, axis=1, keepdims=True)
    lane = jax.lax.broadcasted_iota(jnp.int32, (TILE, KK), 1)
    amax = jnp.min(jnp.where(logits == m, lane, KK), axis=1, keepdims=True)
    onehot = (lane == amax).astype(jnp.float32)
    ri = jax.lax.broadcasted_iota(jnp.int32, (TILE, TILE), 0)
    ci = jax.lax.broadcasted_iota(jnp.int32, (TILE, TILE), 1)
    ltri = (ci < ri).astype(jnp.float32)
    ranks = jax.lax.dot(ltri, onehot,
                        precision=jax.lax.Precision.HIGHEST)
    rank_tok = jnp.sum(ranks * onehot, axis=1, keepdims=True)
    a_ref[...] = amax
    rank_ref[...] = rank_tok
    cnt_ref[...] = jnp.sum(onehot, axis=0, keepdims=True)[None]


def _classify(x, wc, bc):
    return pl.pallas_call(
        _classify_body,
        grid=(NT,),
        in_specs=[
            pl.BlockSpec((TILE, DD), lambda t: (t, 0)),
            pl.BlockSpec((DD, KK), lambda t: (0, 0)),
            pl.BlockSpec((1, KK), lambda t: (0, 0)),
        ],
        out_specs=[
            pl.BlockSpec((TILE, 1), lambda t: (t, 0)),
            pl.BlockSpec((TILE, 1), lambda t: (t, 0)),
            pl.BlockSpec((1, 1, KK), lambda t: (t, 0, 0)),
        ],
        out_shape=[
            jax.ShapeDtypeStruct((BB, 1), jnp.int32),
            jax.ShapeDtypeStruct((BB, 1), jnp.float32),
            jax.ShapeDtypeStruct((NT, 1, KK), jnp.float32),
        ],
    )(x, wc, bc)


def _destiny_body(a_ref, rank_ref, cnt_ref, dest4_ref, offs_ref):
    cnts = cnt_ref[:, 0, :]
    ri = jax.lax.broadcasted_iota(jnp.int32, (NT, NT), 0)
    ci = jax.lax.broadcasted_iota(jnp.int32, (NT, NT), 1)
    ltri = (ci < ri).astype(jnp.float32)
    carry = jax.lax.dot(ltri, cnts,
                        precision=jax.lax.Precision.HIGHEST)
    tot = jnp.sum(cnts, axis=0, keepdims=True)
    er = jax.lax.broadcasted_iota(jnp.int32, (KK, KK), 0)
    ec = jax.lax.broadcasted_iota(jnp.int32, (KK, KK), 1)
    xtri = (er < ec).astype(jnp.float32)
    offs = jax.lax.dot(tot, xtri,
                       precision=jax.lax.Precision.HIGHEST)
    v = offs[None] + carry[:, None, :]
    vtok = jnp.broadcast_to(v, (NT, TILE, KK)).reshape(BB, KK)
    a_t = a_ref[...]
    lane = jax.lax.broadcasted_iota(jnp.int32, (BB, KK), 1)
    onehot = (lane == a_t).astype(jnp.float32)
    base = jnp.sum(onehot * vtok, axis=1, keepdims=True)
    dest = (base + rank_ref[...]).astype(jnp.int32)
    sub = jax.lax.broadcasted_iota(jnp.int32, (BB, SPLIT), 1)
    dest4_ref[...] = dest + sub * BB
    offs_ref[...] = offs.astype(jnp.int32)


def _destiny(a, rank, cnt):
    return pl.pallas_call(
        _destiny_body,
        in_specs=[
            pl.BlockSpec((BB, 1), lambda: (0, 0)),
            pl.BlockSpec((BB, 1), lambda: (0, 0)),
            pl.BlockSpec((NT, 1, KK), lambda: (0, 0, 0)),
        ],
        out_specs=[
            pl.BlockSpec((BB, SPLIT), lambda: (0, 0)),
            pl.BlockSpec((1, KK), lambda: (0, 0)),
        ],
        out_shape=[
            jax.ShapeDtypeStruct((BB, SPLIT), jnp.int32),
            jax.ShapeDtypeStruct((1, KK), jnp.int32),
        ],
    )(a, rank, cnt)


_SC_WIN = 128
_NW = BB // _SC_WIN


def _sc_scatter(x, dest_row):
    mesh = plsc.VectorSubcoreMesh(core_axis_name="core",
                                  subcore_axis_name="subcore")

    @functools.partial(
        pl.kernel,
        out_type=jax.ShapeDtypeStruct((NROWS, SUBD), jnp.float32),
        mesh=mesh)
    def run(x_hbm, i_hbm, o_hbm):
        def body(x_vmem, i_vmem):
            pltpu.sync_copy(x_vmem, o_hbm.at[i_vmem.at[0]])

        pltpu.emit_pipeline(
            body,
            grid=(SPLIT, _NW),
            in_specs=[
                pl.BlockSpec((_SC_WIN, SUBD), lambda c, w: (w, c)),
                pl.BlockSpec((1, _SC_WIN), lambda c, w: (0, c * _NW + w)),
            ],
            out_specs=[],
            core_axis_name=("core", "subcore"),
            dimension_semantics=(pltpu.PARALLEL, pltpu.PARALLEL),
        )(x_hbm, i_hbm)

    return run(x, dest_row)


def _sc_gather(recon4, dest_row):
    mesh = plsc.VectorSubcoreMesh(core_axis_name="core",
                                  subcore_axis_name="subcore")

    @functools.partial(
        pl.kernel,
        out_type=jax.ShapeDtypeStruct((BB, DD), jnp.float32),
        mesh=mesh)
    def run(r_hbm, i_hbm, o_hbm):
        def body(i_vmem, o_vmem):
            pltpu.sync_copy(r_hbm.at[i_vmem.at[0]], o_vmem)

        pltpu.emit_pipeline(
            body,
            grid=(SPLIT, _NW),
            in_specs=[pl.BlockSpec((1, _SC_WIN), lambda c, w: (0, c * _NW + w))],
            out_specs=[pl.BlockSpec((_SC_WIN, SUBD), lambda c, w: (w, c))],
            core_axis_name=("core", "subcore"),
            dimension_semantics=(pltpu.PARALLEL, pltpu.PARALLEL),
        )(i_hbm, o_hbm)

    return run(recon4, dest_row)


def _ae_body(offs_ref, x0_ref, x1_ref, x2_ref, x3_ref,
             w1_ref, b1_ref, w2_ref, b2_ref, o_ref, r_scr):
    t = pl.program_id(0)
    c = pl.program_id(1)
    row0 = t * TILE
    rows = jax.lax.broadcasted_iota(jnp.int32, (TILE, 1), 0)
    x_refs = (x0_ref, x1_ref, x2_ref, x3_ref)

    @pl.when(c == 0)
    def _compute():
        for e in range(KK):
            s = jnp.clip(offs_ref[e] - row0, 0, TILE)
            en = jnp.clip(offs_ref[e + 1] - row0, 0, TILE)

            @pl.when(en > s)
            def _():
                h = jnp.dot(x_refs[0][...], w1_ref[e, 0:HH, :],
                            preferred_element_type=jnp.float32)
                for cc in range(1, SPLIT):
                    h = h + jnp.dot(
                        x_refs[cc][...],
                        w1_ref[e, cc * HH:(cc + 1) * HH, :],
                        preferred_element_type=jnp.float32)
                h = jax.nn.relu(h + b1_ref[e][None, :])
                mask = (rows >= s) & (rows < en)
                for cc in range(SPLIT):
                    r = jnp.dot(h, w2_ref[e, :, cc * SUBD:(cc + 1) * SUBD],
                                preferred_element_type=jnp.float32)
                    r = r + b2_ref[e, cc * SUBD:(cc + 1) * SUBD][None, :]
                    r_scr[cc] = jnp.where(mask, r, r_scr[cc])

    o_ref[...] = r_scr[c]


def _grouped_ae(offs9, xs4, w1, b1, w2, b2):
    grid_spec = pltpu.PrefetchScalarGridSpec(
        num_scalar_prefetch=1,
        grid=(NT, SPLIT),
        in_specs=[
            pl.BlockSpec((TILE, SUBD), lambda t, c, offs: (0 * NT + t, 0)),
            pl.BlockSpec((TILE, SUBD), lambda t, c, offs: (1 * NT + t, 0)),
            pl.BlockSpec((TILE, SUBD), lambda t, c, offs: (2 * NT + t, 0)),
            pl.BlockSpec((TILE, SUBD), lambda t, c, offs: (3 * NT + t, 0)),
            pl.BlockSpec((KK, DD, HH), lambda t, c, offs: (0, 0, 0)),
            pl.BlockSpec((KK, HH), lambda t, c, offs: (0, 0)),
            pl.BlockSpec((KK, HH, DD), lambda t, c, offs: (0, 0, 0)),
            pl.BlockSpec((KK, DD), lambda t, c, offs: (0, 0)),
        ],
        out_specs=pl.BlockSpec((TILE, SUBD), lambda t, c, offs: (c * NT + t, 0)),
        scratch_shapes=[pltpu.VMEM((SPLIT, TILE, SUBD), jnp.float32)],
    )
    return pl.pallas_call(
        _ae_body,
        grid_spec=grid_spec,
        out_shape=jax.ShapeDtypeStruct((NROWS, SUBD), jnp.float32),
    )(offs9, xs4, xs4, xs4, xs4, w1, b1, w2, b2)


def kernel(x, W1, b1, W2, b2, Wc, bc):
    a, rank, cnt = _classify(x, Wc, bc.reshape(1, KK))
    dest4, offs = _destiny(a, rank, cnt)
    offs9 = jnp.concatenate(
        [offs.reshape(KK), jnp.array([BB], jnp.int32)])
    dest_row = dest4.T.reshape(1, NROWS)
    xs4 = _sc_scatter(x, dest_row)
    recon4 = _grouped_ae(offs9, xs4, W1, b1, W2, b2)
    x_out = _sc_gather(recon4, dest_row)
    return (x_out, a.reshape(BB))


# speedup vs baseline: 2.0250x; 1.4130x over previous
import functools

import jax
import jax.numpy as jnp
from jax.experimental import pallas as pl
from jax.experimental.pallas import tpu as pltpu
from jax.experimental.pallas import tpu_sc as plsc

BB, DD, HH, KK = 8192, 1024, 256, 8
TILE = 256
NT = BB // TILE
TILEC = 1024
NTC = BB // TILEC
SPLIT = 4
SUBD = DD // SPLIT
NROWS = BB * SPLIT


def _classify_body(x_ref, wc_ref, bc_ref, a_ref, rank_ref, cnt_ref):
    x_t = x_ref[...]
    logits = jnp.dot(x_t, wc_ref[...], preferred_element_type=jnp.float32)
    logits = logits + bc_ref[...]
    m = jnp.max(logits, axis=1, keepdims=True)
    lane = jax.lax.broadcasted_iota(jnp.int32, (TILEC, KK), 1)
    amax = jnp.min(jnp.where(logits == m, lane, KK), axis=1, keepdims=True)
    onehot = (lane == amax).astype(jnp.float32)
    ri = jax.lax.broadcasted_iota(jnp.int32, (TILEC, TILEC), 0)
    ci = jax.lax.broadcasted_iota(jnp.int32, (TILEC, TILEC), 1)
    lbd = ((ci < ri) & (ci // TILE == ri // TILE)).astype(jnp.float32)
    ranks = jax.lax.dot(lbd, onehot)
    csum = jnp.sum(onehot.reshape(TILEC // TILE, TILE, KK), axis=1)
    cr = jax.lax.broadcasted_iota(jnp.int32, (TILEC // TILE,) * 2, 0)
    cc = jax.lax.broadcasted_iota(jnp.int32, (TILEC // TILE,) * 2, 1)
    lt4 = (cc < cr).astype(jnp.float32)
    cbase = jax.lax.dot(lt4, csum)
    base_tok = jnp.broadcast_to(cbase[:, None, :],
                                (TILEC // TILE, TILE, KK)).reshape(TILEC, KK)
    rank_tok = jnp.sum((ranks + base_tok) * onehot, axis=1, keepdims=True)
    a_ref[...] = amax
    rank_ref[...] = rank_tok
    cnt_ref[...] = jnp.sum(onehot, axis=0, keepdims=True)[None]


def _classify(x, wc, bc):
    return pl.pallas_call(
        _classify_body,
        grid=(NTC,),
        in_specs=[
            pl.BlockSpec((TILEC, DD), lambda t: (t, 0)),
            pl.BlockSpec((DD, KK), lambda t: (0, 0)),
            pl.BlockSpec((1, KK), lambda t: (0, 0)),
        ],
        out_specs=[
            pl.BlockSpec((TILEC, 1), lambda t: (t, 0)),
            pl.BlockSpec((TILEC, 1), lambda t: (t, 0)),
            pl.BlockSpec((1, 1, KK), lambda t: (t, 0, 0)),
        ],
        out_shape=[
            jax.ShapeDtypeStruct((BB, 1), jnp.int32),
            jax.ShapeDtypeStruct((BB, 1), jnp.float32),
            jax.ShapeDtypeStruct((NTC, 1, KK), jnp.float32),
        ],
    )(x, wc, bc)


def _destiny_body(a_ref, rank_ref, cnt_ref, dest4_ref, offs_ref):
    cnts = cnt_ref[:, 0, :]
    ri = jax.lax.broadcasted_iota(jnp.int32, (NTC, NTC), 0)
    ci = jax.lax.broadcasted_iota(jnp.int32, (NTC, NTC), 1)
    ltri = (ci < ri).astype(jnp.float32)
    carry = jax.lax.dot(ltri, cnts,
                        precision=jax.lax.Precision.HIGHEST)
    tot = jnp.sum(cnts, axis=0, keepdims=True)
    er = jax.lax.broadcasted_iota(jnp.int32, (KK, KK), 0)
    ec = jax.lax.broadcasted_iota(jnp.int32, (KK, KK), 1)
    xtri = (er < ec).astype(jnp.float32)
    offs = jax.lax.dot(tot, xtri,
                       precision=jax.lax.Precision.HIGHEST)
    v = offs[None] + carry[:, None, :]
    vtok = jnp.broadcast_to(v, (NTC, TILEC, KK)).reshape(BB, KK)
    a_t = a_ref[...]
    lane = jax.lax.broadcasted_iota(jnp.int32, (BB, KK), 1)
    onehot = (lane == a_t).astype(jnp.float32)
    base = jnp.sum(onehot * vtok, axis=1, keepdims=True)
    dest = (base + rank_ref[...]).astype(jnp.int32)
    sub = jax.lax.broadcasted_iota(jnp.int32, (BB, SPLIT), 1)
    dest4_ref[...] = dest + sub * BB
    offs_ref[...] = offs.astype(jnp.int32)


def _destiny(a, rank, cnt):
    return pl.pallas_call(
        _destiny_body,
        in_specs=[
            pl.BlockSpec((BB, 1), lambda: (0, 0)),
            pl.BlockSpec((BB, 1), lambda: (0, 0)),
            pl.BlockSpec((NTC, 1, KK), lambda: (0, 0, 0)),
        ],
        out_specs=[
            pl.BlockSpec((BB, SPLIT), lambda: (0, 0)),
            pl.BlockSpec((1, KK), lambda: (0, 0)),
        ],
        out_shape=[
            jax.ShapeDtypeStruct((BB, SPLIT), jnp.int32),
            jax.ShapeDtypeStruct((1, KK), jnp.int32),
        ],
    )(a, rank, cnt)


_SC_WIN = 128
_NW = BB // _SC_WIN


def _sc_scatter(x, dest_row):
    mesh = plsc.VectorSubcoreMesh(core_axis_name="core",
                                  subcore_axis_name="subcore")

    @functools.partial(
        pl.kernel,
        out_type=jax.ShapeDtypeStruct((NROWS, SUBD), jnp.float32),
        mesh=mesh)
    def run(x_hbm, i_hbm, o_hbm):
        def body(x_vmem, i_vmem):
            pltpu.sync_copy(x_vmem, o_hbm.at[i_vmem.at[0]])

        pltpu.emit_pipeline(
            body,
            grid=(SPLIT, _NW),
            in_specs=[
                pl.BlockSpec((_SC_WIN, SUBD), lambda c, w: (w, c)),
                pl.BlockSpec((1, _SC_WIN), lambda c, w: (0, c * _NW + w)),
            ],
            out_specs=[],
            core_axis_name=("core", "subcore"),
            dimension_semantics=(pltpu.PARALLEL, pltpu.PARALLEL),
        )(x_hbm, i_hbm)

    return run(x, dest_row)


def _sc_gather(r0, r1, r2, r3, dest_plain):
    mesh = plsc.VectorSubcoreMesh(core_axis_name="core",
                                  subcore_axis_name="subcore")

    @functools.partial(
        pl.kernel,
        out_type=jax.ShapeDtypeStruct((BB, DD), jnp.float32),
        mesh=mesh)
    def run(r0_hbm, r1_hbm, r2_hbm, r3_hbm, i_hbm, o_hbm):
        for cplane, r_hbm in enumerate((r0_hbm, r1_hbm, r2_hbm, r3_hbm)):
            def body(i_vmem, o_vmem, r_hbm=r_hbm):
                pltpu.sync_copy(r_hbm.at[i_vmem.at[0]], o_vmem)

            pltpu.emit_pipeline(
                body,
                grid=(_NW,),
                in_specs=[pl.BlockSpec((1, _SC_WIN), lambda w: (0, w))],
                out_specs=[pl.BlockSpec((_SC_WIN, SUBD),
                                        lambda w, cplane=cplane: (w, cplane))],
                core_axis_name=("core", "subcore"),
                dimension_semantics=(pltpu.PARALLEL,),
            )(i_hbm, o_hbm)

    return run(r0, r1, r2, r3, dest_plain)


def _ae_body(offs_ref, x0_ref, x1_ref, x2_ref, x3_ref,
             w1_ref, b1_ref, w2_ref, b2_ref,
             r0_ref, r1_ref, r2_ref, r3_ref):
    t = pl.program_id(0)
    row0 = t * TILE
    rows = jax.lax.broadcasted_iota(jnp.int32, (TILE, 1), 0)
    x_cat = jnp.concatenate(
        [x0_ref[...], x1_ref[...], x2_ref[...], x3_ref[...]],
        axis=1)
    r_refs = (r0_ref, r1_ref, r2_ref, r3_ref)
    for e in range(KK):
        s = jnp.clip(offs_ref[e] - row0, 0, TILE)
        en = jnp.clip(offs_ref[e + 1] - row0, 0, TILE)

        @pl.when(en > s)
        def _():
            h = jnp.dot(x_cat, w1_ref[e], preferred_element_type=jnp.float32)
            h = jax.nn.relu(h + b1_ref[e][None, :])
            r = jnp.dot(h, w2_ref[e], preferred_element_type=jnp.float32)
            r = r + b2_ref[e][None, :]
            mask = (rows >= s) & (rows < en)
            for cp in range(SPLIT):
                r_refs[cp][...] = jnp.where(
                    mask, r[:, cp * SUBD:(cp + 1) * SUBD], r_refs[cp][...])


def _grouped_ae(offs9, xs4, w1, b1, w2, b2):
    grid_spec = pltpu.PrefetchScalarGridSpec(
        num_scalar_prefetch=1,
        grid=(NT,),
        in_specs=[
            pl.BlockSpec((TILE, SUBD), lambda t, offs: (0 * NT + t, 0)),
            pl.BlockSpec((TILE, SUBD), lambda t, offs: (1 * NT + t, 0)),
            pl.BlockSpec((TILE, SUBD), lambda t, offs: (2 * NT + t, 0)),
            pl.BlockSpec((TILE, SUBD), lambda t, offs: (3 * NT + t, 0)),
            pl.BlockSpec((KK, DD, HH), lambda t, offs: (0, 0, 0)),
            pl.BlockSpec((KK, HH), lambda t, offs: (0, 0)),
            pl.BlockSpec((KK, HH, DD), lambda t, offs: (0, 0, 0)),
            pl.BlockSpec((KK, DD), lambda t, offs: (0, 0)),
        ],
        out_specs=[
            pl.BlockSpec((TILE, SUBD), lambda t, offs: (t, 0)),
            pl.BlockSpec((TILE, SUBD), lambda t, offs: (t, 0)),
            pl.BlockSpec((TILE, SUBD), lambda t, offs: (t, 0)),
            pl.BlockSpec((TILE, SUBD), lambda t, offs: (t, 0)),
        ],
    )
    return pl.pallas_call(
        _ae_body,
        grid_spec=grid_spec,
        out_shape=[jax.ShapeDtypeStruct((BB, SUBD), jnp.float32)] * SPLIT,
    )(offs9, xs4, xs4, xs4, xs4, w1, b1, w2, b2)


def kernel(x, W1, b1, W2, b2, Wc, bc):
    a, rank, cnt = _classify(x, Wc, bc.reshape(1, KK))
    dest4, offs = _destiny(a, rank, cnt)
    offs9 = jnp.concatenate(
        [offs.reshape(KK), jnp.array([BB], jnp.int32)])
    dest_row = dest4.T.reshape(1, NROWS)
    dest_plain = dest4[:, 0].reshape(1, BB)
    xs4 = _sc_scatter(x, dest_row)
    r0, r1, r2, r3 = _grouped_ae(offs9, xs4, W1, b1, W2, b2)
    x_out = _sc_gather(r0, r1, r2, r3, dest_plain)
    return (x_out, a.reshape(BB))
